# Initial kernel scaffold; baseline (speedup 1.0000x reference)
import jax
import jax.numpy as jnp
from jax.experimental import pallas as pl


def kernel(x, edge_index, edge_weight, batch, W1_self, b1_self, W1_s2d, b1_s2d, W1_d2s, b1_d2s, W2_self, b2_self, W2_s2d, b2_s2d, W2_d2s, b2_d2s, Wl1, bl1, Wl2, bl2):
    raise NotImplementedError("write your pallas kernel here")


# speedup vs baseline: 2.8268x; 2.8268x over previous
import functools

import jax
import jax.numpy as jnp
from jax import lax
from jax.experimental import pallas as pl
from jax.experimental.pallas import tpu as pltpu
from jax.experimental.pallas import tpu_sc as plsc

N = 10000
NPAD = 10240
E = 160000
H = 256
HH = 128
G = 64
BLK = 1024
GRID = NPAD // BLK
NTILES = 16
K = 80
EPT = E // NTILES
CPT = EPT // K
RPT = NPAD // NTILES


def _tc1_body(x_ref, w_ref, s_ref, p_ref, q_ref):
    r = jnp.dot(x_ref[...], w_ref[...], preferred_element_type=jnp.float32)
    s_ref[...] = r[:, :H]
    p_ref[...] = jnp.stack([r[:, H:H + HH], r[:, H + HH:H + 2 * HH]])
    q_ref[...] = jnp.stack([r[:, H + 2 * HH:H + 3 * HH], r[:, H + 3 * HH:]])


def _tc_transform(xp, wcat):
    return pl.pallas_call(
        _tc1_body,
        grid=(GRID,),
        in_specs=[
            pl.BlockSpec((BLK, H), lambda i: (i, 0)),
            pl.BlockSpec((H, 3 * H), lambda i: (0, 0)),
        ],
        out_specs=[
            pl.BlockSpec((BLK, H), lambda i: (i, 0)),
            pl.BlockSpec((2, BLK, HH), lambda i: (0, i, 0)),
            pl.BlockSpec((2, BLK, HH), lambda i: (0, i, 0)),
        ],
        out_shape=[
            jax.ShapeDtypeStruct((NPAD, H), jnp.float32),
            jax.ShapeDtypeStruct((2, NPAD, HH), jnp.float32),
            jax.ShapeDtypeStruct((2, NPAD, HH), jnp.float32),
        ],
    )(xp, wcat)


def _sc_agg_body(src_h, dst_h, p_h, q_h, sump_h, sumq_h, cnt_h,
                 gidx, gidx2, sidx, rows, onesv, zvec, zbuf, acc, cacc, sem):
    c = lax.axis_index("c")
    s = lax.axis_index("s")
    row_off = c * NPAD
    base = s * RPT
    ebase = s * EPT

    def _zb(r, _):
        for j in range(HH // 16):
            zbuf[r, pl.ds(j * 16, 16)] = jnp.zeros((16,), jnp.float32)
        return 0
    lax.fori_loop(0, K, _zb, 0)

    def _zc(j, _):
        zvec[pl.ds(j * 16, 16)] = jnp.zeros((16,), jnp.float32)
        return 0
    lax.fori_loop(0, RPT // 16, _zc, 0)
    for j in range(K // 16):
        onesv[pl.ds(j * 16, 16)] = jnp.ones((16,), jnp.float32)

    def _zero_acc():
        for kk in range(RPT // K):
            pltpu.sync_copy(zbuf, acc.at[pl.ds(base + kk * K, K), :])

    def _run_phase(table_h, gsrc_h, ssrc_h, do_cnt):
        def body(i, carry):
            b = ebase + i * K
            pltpu.sync_copy(gsrc_h.at[pl.ds(b, K)], gidx)
            pltpu.sync_copy(ssrc_h.at[pl.ds(b, K)], sidx)
            off = jnp.full((16,), row_off, jnp.int32)
            for j in range(K // 16):
                gidx2[pl.ds(j * 16, 16)] = gidx[pl.ds(j * 16, 16)] + off
            pltpu.async_copy(table_h.at[gidx2], rows, sem).wait()
            pltpu.sync_copy(rows, acc.at[sidx], add=True)
            if do_cnt:
                @pl.when(c == 0)
                def _():
                    pltpu.sync_copy(onesv, cacc.at[sidx], add=True)

                @pl.when(c == 1)
                def _():
                    pltpu.sync_copy(onesv, cacc.at[gidx], add=True)
            return carry
        lax.fori_loop(0, CPT, body, 0)

    _zero_acc()
    pltpu.sync_copy(zvec, cacc.at[pl.ds(base, RPT)])
    plsc.subcore_barrier()
    _run_phase(p_h, src_h, dst_h, True)
    plsc.subcore_barrier()
    pltpu.sync_copy(acc.at[pl.ds(base, RPT), :],
                    sump_h.at[c, pl.ds(base, RPT), :])
    pltpu.sync_copy(cacc.at[pl.ds(base, RPT)], cnt_h.at[c, pl.ds(base, RPT)])

    _zero_acc()
    plsc.subcore_barrier()
    _run_phase(q_h, dst_h, src_h, False)
    plsc.subcore_barrier()
    pltpu.sync_copy(acc.at[pl.ds(base, RPT), :],
                    sumq_h.at[c, pl.ds(base, RPT), :])


def _sc_agg(src, dst, p2d, q2d):
    mesh = plsc.VectorSubcoreMesh(core_axis_name="c", subcore_axis_name="s")
    f = pl.kernel(
        _sc_agg_body,
        out_type=[
            jax.ShapeDtypeStruct((2, NPAD, HH), jnp.float32),
            jax.ShapeDtypeStruct((2, NPAD, HH), jnp.float32),
            jax.ShapeDtypeStruct((2, NPAD), jnp.float32),
        ],
        mesh=mesh,
        scratch_types=[
            pltpu.VMEM((K,), jnp.int32),
            pltpu.VMEM((K,), jnp.int32),
            pltpu.VMEM((K,), jnp.int32),
            pltpu.VMEM((K, HH), jnp.float32),
            pltpu.VMEM((K,), jnp.float32),
            pltpu.VMEM((RPT,), jnp.float32),
            pltpu.VMEM((K, HH), jnp.float32),
            pltpu.VMEM_SHARED((NPAD, HH), jnp.float32),
            pltpu.VMEM_SHARED((NPAD,), jnp.float32),
            pltpu.SemaphoreType.DMA,
        ],
    )
    return f(src, dst, p2d, q2d)


def _inv_counts(cd_ref, cs_ref):
    invd = 1.0 / jnp.maximum(cd_ref[...], 1.0)
    invs = 1.0 / jnp.maximum(cs_ref[...], 1.0)
    return invd, invs


def _combine(s_ref, sp_ref, sq_ref, b_ref, invd, invs):
    sp = jnp.concatenate([sp_ref[0], sp_ref[1]], axis=1)
    sq = jnp.concatenate([sq_ref[0], sq_ref[1]], axis=1)
    return s_ref[...] + sp * invd + sq * invs + b_ref[...]


def _tc2_body(s_ref, sp_ref, sq_ref, cd_ref, cs_ref, b_ref, w_ref,
              s2_ref, p2_ref, q2_ref):
    invd, invs = _inv_counts(cd_ref, cs_ref)
    h = jnp.maximum(_combine(s_ref, sp_ref, sq_ref, b_ref, invd, invs), 0.0)
    r = jnp.dot(h, w_ref[...], preferred_element_type=jnp.float32)
    s2_ref[...] = r[:, :H]
    p2_ref[...] = jnp.stack([r[:, H:H + HH], r[:, H + HH:H + 2 * HH]])
    q2_ref[...] = jnp.stack([r[:, H + 2 * HH:H + 3 * HH], r[:, H + 3 * HH:]])


def _tc_combine(s1, sump, sumq, cntd, cnts, b1c, wcat2):
    return pl.pallas_call(
        _tc2_body,
        grid=(GRID,),
        in_specs=[
            pl.BlockSpec((BLK, H), lambda i: (i, 0)),
            pl.BlockSpec((2, BLK, HH), lambda i: (0, i, 0)),
            pl.BlockSpec((2, BLK, HH), lambda i: (0, i, 0)),
            pl.BlockSpec((BLK, 1), lambda i: (i, 0)),
            pl.BlockSpec((BLK, 1), lambda i: (i, 0)),
            pl.BlockSpec((1, H), lambda i: (0, 0)),
            pl.BlockSpec((H, 3 * H), lambda i: (0, 0)),
        ],
        out_specs=[
            pl.BlockSpec((BLK, H), lambda i: (i, 0)),
            pl.BlockSpec((2, BLK, HH), lambda i: (0, i, 0)),
            pl.BlockSpec((2, BLK, HH), lambda i: (0, i, 0)),
        ],
        out_shape=[
            jax.ShapeDtypeStruct((NPAD, H), jnp.float32),
            jax.ShapeDtypeStruct((2, NPAD, HH), jnp.float32),
            jax.ShapeDtypeStruct((2, NPAD, HH), jnp.float32),
        ],
    )(s1, sump, sumq, cntd, cnts, b1c, wcat2)


def _tc3_body(s_ref, sp_ref, sq_ref, cd_ref, cs_ref, b_ref, batch_ref,
              wl1_ref, bl1_ref, wl2_ref, bl2_ref, out_ref, pool_ref):
    i = pl.program_id(0)
    invd, invs = _inv_counts(cd_ref, cs_ref)
    h = _combine(s_ref, sp_ref, sq_ref, b_ref, invd, invs)

    @pl.when(i == 0)
    def _():
        pool_ref[...] = jnp.full((G, H), -jnp.inf, jnp.float32)

    bslice = batch_ref[...]
    glo = jnp.min(bslice)
    ghi = jnp.minimum(jnp.max(bslice), G - 1)

    def seg(g, carry):
        m = bslice == g
        v = jnp.max(jnp.where(m, h, -jnp.inf), axis=0, keepdims=True)
        sel = lax.broadcasted_iota(jnp.int32, (G, 1), 0) == g
        pool_ref[...] = jnp.maximum(pool_ref[...],
                                    jnp.where(sel, v, -jnp.inf))
        return carry
    lax.fori_loop(glo, ghi + 1, seg, 0)

    @pl.when(i == GRID - 1)
    def _():
        p = pool_ref[...]
        t = jnp.maximum(
            jnp.dot(p, wl1_ref[...], preferred_element_type=jnp.float32)
            + bl1_ref[...], 0.0)
        out_ref[...] = (
            jnp.dot(t, wl2_ref[...], preferred_element_type=jnp.float32)
            + bl2_ref[...])


def _tc_final(s2, sump, sumq, cntd, cnts, b2c, batchp, wl1p, bl1p, wl2p, bl2p):
    return pl.pallas_call(
        _tc3_body,
        grid=(GRID,),
        in_specs=[
            pl.BlockSpec((BLK, H), lambda i: (i, 0)),
            pl.BlockSpec((2, BLK, HH), lambda i: (0, i, 0)),
            pl.BlockSpec((2, BLK, HH), lambda i: (0, i, 0)),
            pl.BlockSpec((BLK, 1), lambda i: (i, 0)),
            pl.BlockSpec((BLK, 1), lambda i: (i, 0)),
            pl.BlockSpec((1, H), lambda i: (0, 0)),
            pl.BlockSpec((BLK, 1), lambda i: (i, 0)),
            pl.BlockSpec((H, 128), lambda i: (0, 0)),
            pl.BlockSpec((1, 128), lambda i: (0, 0)),
            pl.BlockSpec((128, 128), lambda i: (0, 0)),
            pl.BlockSpec((1, 128), lambda i: (0, 0)),
        ],
        out_specs=pl.BlockSpec((G, 128), lambda i: (0, 0)),
        out_shape=jax.ShapeDtypeStruct((G, 128), jnp.float32),
        scratch_shapes=[pltpu.VMEM((G, H), jnp.float32)],
    )(s2, sump, sumq, cntd, cnts, b2c, batchp, wl1p, bl1p, wl2p, bl2p)


def kernel(x, edge_index, edge_weight, batch,
           W1_self, b1_self, W1_s2d, b1_s2d, W1_d2s, b1_d2s,
           W2_self, b2_self, W2_s2d, b2_s2d, W2_d2s, b2_d2s,
           Wl1, bl1, Wl2, bl2):
    src = edge_index[0]
    dst = edge_index[1]
    wcat1 = jnp.concatenate([W1_self, 0.5 * W1_s2d, 0.5 * W1_d2s], axis=1)
    b1c = (b1_self + 0.5 * b1_s2d + 0.5 * b1_d2s).reshape(1, H)
    wcat2 = jnp.concatenate([W2_self, 0.5 * W2_s2d, 0.5 * W2_d2s], axis=1)
    b2c = (b2_self + 0.5 * b2_s2d + 0.5 * b2_d2s).reshape(1, H)
    xp = jnp.pad(x, ((0, NPAD - N), (0, 0)))
    batchp = jnp.pad(batch, (0, NPAD - N), constant_values=G).reshape(NPAD, 1)
    wl1p = jnp.pad(Wl1, ((0, 0), (0, 128 - Wl1.shape[1])))
    bl1p = jnp.pad(bl1, (0, 128 - bl1.shape[0])).reshape(1, 128)
    wl2p = jnp.pad(Wl2, ((0, 128 - Wl2.shape[0]), (0, 128 - Wl2.shape[1])))
    bl2p = jnp.pad(bl2, (0, 128 - bl2.shape[0])).reshape(1, 128)

    s1, p1, q1 = _tc_transform(xp, wcat1)
    sump1, sumq1, cnt = _sc_agg(src, dst,
                                p1.reshape(2 * NPAD, HH),
                                q1.reshape(2 * NPAD, HH))
    cntd = cnt[0].reshape(NPAD, 1)
    cnts = cnt[1].reshape(NPAD, 1)
    s2, p2, q2 = _tc_combine(s1, sump1, sumq1, cntd, cnts, b1c, wcat2)
    sump2, sumq2, _ = _sc_agg(src, dst,
                              p2.reshape(2 * NPAD, HH),
                              q2.reshape(2 * NPAD, HH))
    out = _tc_final(s2, sump2, sumq2, cntd, cnts, b2c, batchp,
                    wl1p, bl1p, wl2p, bl2p)
    return out[:, :1]


# speedup vs baseline: 5.6082x; 1.9839x over previous
import functools

import jax
import jax.numpy as jnp
from jax import lax
from jax.experimental import pallas as pl
from jax.experimental.pallas import tpu as pltpu
from jax.experimental.pallas import tpu_sc as plsc

N = 10000
NPAD = 10240
E = 160000
H = 256
HH = 128
G = 64
BLK = 1024
GRID = NPAD // BLK
NTILES = 16
K = 80
EPT = E // NTILES
CPT = EPT // K
RPT = NPAD // NTILES


def _tc1_body(x_ref, w_ref, s_ref, p_ref, q_ref):
    r = jnp.dot(x_ref[...], w_ref[...], preferred_element_type=jnp.float32)
    s_ref[...] = r[:, :H]
    p_ref[...] = jnp.stack([r[:, H:H + HH], r[:, H + HH:H + 2 * HH]])
    q_ref[...] = jnp.stack([r[:, H + 2 * HH:H + 3 * HH], r[:, H + 3 * HH:]])


def _tc_transform(xp, wcat):
    return pl.pallas_call(
        _tc1_body,
        grid=(GRID,),
        in_specs=[
            pl.BlockSpec((BLK, H), lambda i: (i, 0)),
            pl.BlockSpec((H, 3 * H), lambda i: (0, 0)),
        ],
        out_specs=[
            pl.BlockSpec((BLK, H), lambda i: (i, 0)),
            pl.BlockSpec((2, BLK, HH), lambda i: (0, i, 0)),
            pl.BlockSpec((2, BLK, HH), lambda i: (0, i, 0)),
        ],
        out_shape=[
            jax.ShapeDtypeStruct((NPAD, H), jnp.float32),
            jax.ShapeDtypeStruct((2, NPAD, HH), jnp.float32),
            jax.ShapeDtypeStruct((2, NPAD, HH), jnp.float32),
        ],
    )(xp, wcat)


def _sc_agg_body(do_cnt, src_h, dst_h, p_h, q_h, sump_h, sumq_h, cnt_h,
                 gbuf0, gbuf1, sbuf0, sbuf1, gadj0, gadj1, rows0, rows1,
                 onesv, zvec, acc, cacc, sem_g, sem_i):
    gbuf = (gbuf0, gbuf1)
    sbuf = (sbuf0, sbuf1)
    gadj = (gadj0, gadj1)
    rows = (rows0, rows1)
    c = lax.axis_index("c")
    s = lax.axis_index("s")
    row_off = c * NPAD
    base = s * RPT

    def _zero_rows0():
        def _zb(r, _):
            for j in range(HH // 16):
                rows[0][r, pl.ds(j * 16, 16)] = jnp.zeros((16,), jnp.float32)
            return 0
        lax.fori_loop(0, K, _zb, 0)

    def _zc(j, _):
        zvec[pl.ds(j * 16, 16)] = jnp.zeros((16,), jnp.float32)
        return 0
    lax.fori_loop(0, RPT // 16, _zc, 0)
    for j in range(K // 16):
        onesv[pl.ds(j * 16, 16)] = jnp.ones((16,), jnp.float32)

    def _zero_acc():
        for kk in range(RPT // K):
            pltpu.sync_copy(rows[0], acc.at[pl.ds(base + kk * K, K), :])

    def _run_phase(table_h, g_h, s_h, cnt_this):
        off = jnp.full((16,), row_off, jnp.int32)

        ebase = s * EPT

        def _load_idx(i, b):
            pltpu.async_copy(g_h.at[pl.ds(ebase + i * K, K)], gbuf[b], sem_i)
            pltpu.async_copy(s_h.at[pl.ds(ebase + i * K, K)], sbuf[b], sem_i)

        def _wait_idx(i, b):
            pltpu.make_async_copy(
                g_h.at[pl.ds(ebase + i * K, K)], gbuf[b], sem_i).wait()
            pltpu.make_async_copy(
                s_h.at[pl.ds(ebase + i * K, K)], sbuf[b], sem_i).wait()

        def _adj_and_gather(i, b):
            for j in range(K // 16):
                gadj[b][pl.ds(j * 16, 16)] = (
                    gbuf[b][pl.ds(j * 16, 16)] + off)
            pltpu.async_copy(table_h.at[gadj[b]], rows[b], sem_g)

        def _step(j, b, last):
            if not last:
                _wait_idx(j + 1, 1 - b)
                _adj_and_gather(j + 1, 1 - b)
            pltpu.make_async_copy(table_h.at[gadj[b]], rows[b], sem_g).wait()
            pltpu.sync_copy(rows[b], acc.at[sbuf[b]], add=True)
            if cnt_this:
                @pl.when(c == 0)
                def _():
                    pltpu.sync_copy(onesv, cacc.at[sbuf[b]], add=True)

                @pl.when(c == 1)
                def _():
                    pltpu.sync_copy(onesv, cacc.at[gbuf[b]], add=True)
            if not last:
                nxt = jnp.minimum(j + 2, CPT - 1)
                _load_idx(nxt, b)
            else:
                _wait_idx(j, 1 - b)

        _load_idx(0, 0)
        _wait_idx(0, 0)
        _load_idx(1, 1)
        _adj_and_gather(0, 0)

        def _pair(j, carry):
            _step(2 * j, 0, False)
            _step(2 * j + 1, 1, False)
            return carry
        lax.fori_loop(0, (CPT - 1) // 2, _pair, 0)
        _step(CPT - 1, 0, True)

    _zero_rows0()
    _zero_acc()
    if do_cnt:
        pltpu.sync_copy(zvec, cacc.at[pl.ds(base, RPT)])
    plsc.subcore_barrier()
    _run_phase(p_h, src_h, dst_h, do_cnt)
    plsc.subcore_barrier()
    pltpu.sync_copy(acc.at[pl.ds(base, RPT), :],
                    sump_h.at[c, pl.ds(base, RPT), :])
    if do_cnt:
        pltpu.sync_copy(cacc.at[pl.ds(base, RPT)],
                        cnt_h.at[c, pl.ds(base, RPT)])

    _zero_rows0()
    _zero_acc()
    plsc.subcore_barrier()
    _run_phase(q_h, dst_h, src_h, False)
    plsc.subcore_barrier()
    pltpu.sync_copy(acc.at[pl.ds(base, RPT), :],
                    sumq_h.at[c, pl.ds(base, RPT), :])


def _sc_agg(src3, dst3, p2d, q2d, do_cnt):
    mesh = plsc.VectorSubcoreMesh(core_axis_name="c", subcore_axis_name="s")
    f = pl.kernel(
        functools.partial(_sc_agg_body, do_cnt),
        out_type=[
            jax.ShapeDtypeStruct((2, NPAD, HH), jnp.float32),
            jax.ShapeDtypeStruct((2, NPAD, HH), jnp.float32),
            jax.ShapeDtypeStruct((2, NPAD), jnp.float32),
        ],
        mesh=mesh,
        scratch_types=[
            pltpu.VMEM((K,), jnp.int32),
            pltpu.VMEM((K,), jnp.int32),
            pltpu.VMEM((K,), jnp.int32),
            pltpu.VMEM((K,), jnp.int32),
            pltpu.VMEM((K,), jnp.int32),
            pltpu.VMEM((K,), jnp.int32),
            pltpu.VMEM((K, HH), jnp.float32),
            pltpu.VMEM((K, HH), jnp.float32),
            pltpu.VMEM((K,), jnp.float32),
            pltpu.VMEM((RPT,), jnp.float32),
            pltpu.VMEM_SHARED((NPAD, HH), jnp.float32),
            pltpu.VMEM_SHARED((NPAD,), jnp.float32),
            pltpu.SemaphoreType.DMA,
            pltpu.SemaphoreType.DMA,
        ],
    )
    return f(src3, dst3, p2d, q2d)


def _inv_counts(cd_ref, cs_ref):
    invd = 1.0 / jnp.maximum(cd_ref[...], 1.0)
    invs = 1.0 / jnp.maximum(cs_ref[...], 1.0)
    return invd, invs


def _combine(s_ref, sp_ref, sq_ref, b_ref, invd, invs):
    sp = jnp.concatenate([sp_ref[0], sp_ref[1]], axis=1)
    sq = jnp.concatenate([sq_ref[0], sq_ref[1]], axis=1)
    return s_ref[...] + sp * invd + sq * invs + b_ref[...]


def _tc2_body(s_ref, sp_ref, sq_ref, cd_ref, cs_ref, b_ref, w_ref,
              s2_ref, p2_ref, q2_ref):
    invd, invs = _inv_counts(cd_ref, cs_ref)
    h = jnp.maximum(_combine(s_ref, sp_ref, sq_ref, b_ref, invd, invs), 0.0)
    r = jnp.dot(h, w_ref[...], preferred_element_type=jnp.float32)
    s2_ref[...] = r[:, :H]
    p2_ref[...] = jnp.stack([r[:, H:H + HH], r[:, H + HH:H + 2 * HH]])
    q2_ref[...] = jnp.stack([r[:, H + 2 * HH:H + 3 * HH], r[:, H + 3 * HH:]])


def _tc_combine(s1, sump, sumq, cntd, cnts, b1c, wcat2):
    return pl.pallas_call(
        _tc2_body,
        grid=(GRID,),
        in_specs=[
            pl.BlockSpec((BLK, H), lambda i: (i, 0)),
            pl.BlockSpec((2, BLK, HH), lambda i: (0, i, 0)),
            pl.BlockSpec((2, BLK, HH), lambda i: (0, i, 0)),
            pl.BlockSpec((BLK, 1), lambda i: (i, 0)),
            pl.BlockSpec((BLK, 1), lambda i: (i, 0)),
            pl.BlockSpec((1, H), lambda i: (0, 0)),
            pl.BlockSpec((H, 3 * H), lambda i: (0, 0)),
        ],
        out_specs=[
            pl.BlockSpec((BLK, H), lambda i: (i, 0)),
            pl.BlockSpec((2, BLK, HH), lambda i: (0, i, 0)),
            pl.BlockSpec((2, BLK, HH), lambda i: (0, i, 0)),
        ],
        out_shape=[
            jax.ShapeDtypeStruct((NPAD, H), jnp.float32),
            jax.ShapeDtypeStruct((2, NPAD, HH), jnp.float32),
            jax.ShapeDtypeStruct((2, NPAD, HH), jnp.float32),
        ],
    )(s1, sump, sumq, cntd, cnts, b1c, wcat2)


def _tc3_body(s_ref, sp_ref, sq_ref, cd_ref, cs_ref, b_ref, batch_ref,
              wl1_ref, bl1_ref, wl2_ref, bl2_ref, out_ref, pool_ref):
    i = pl.program_id(0)
    invd, invs = _inv_counts(cd_ref, cs_ref)
    h = _combine(s_ref, sp_ref, sq_ref, b_ref, invd, invs)

    @pl.when(i == 0)
    def _():
        pool_ref[...] = jnp.full((G, H), -jnp.inf, jnp.float32)

    bslice = batch_ref[...]
    glo = jnp.min(bslice)
    ghi = jnp.minimum(jnp.max(bslice), G - 1)

    def seg(g, carry):
        m = bslice == g
        v = jnp.max(jnp.where(m, h, -jnp.inf), axis=0, keepdims=True)
        sel = lax.broadcasted_iota(jnp.int32, (G, 1), 0) == g
        pool_ref[...] = jnp.maximum(pool_ref[...],
                                    jnp.where(sel, v, -jnp.inf))
        return carry
    lax.fori_loop(glo, ghi + 1, seg, 0)

    @pl.when(i == GRID - 1)
    def _():
        p = pool_ref[...]
        t = jnp.maximum(
            jnp.dot(p, wl1_ref[...], preferred_element_type=jnp.float32)
            + bl1_ref[...], 0.0)
        out_ref[...] = (
            jnp.dot(t, wl2_ref[...], preferred_element_type=jnp.float32)
            + bl2_ref[...])


def _tc_final(s2, sump, sumq, cntd, cnts, b2c, batchp, wl1p, bl1p, wl2p, bl2p):
    return pl.pallas_call(
        _tc3_body,
        grid=(GRID,),
        in_specs=[
            pl.BlockSpec((BLK, H), lambda i: (i, 0)),
            pl.BlockSpec((2, BLK, HH), lambda i: (0, i, 0)),
            pl.BlockSpec((2, BLK, HH), lambda i: (0, i, 0)),
            pl.BlockSpec((BLK, 1), lambda i: (i, 0)),
            pl.BlockSpec((BLK, 1), lambda i: (i, 0)),
            pl.BlockSpec((1, H), lambda i: (0, 0)),
            pl.BlockSpec((BLK, 1), lambda i: (i, 0)),
            pl.BlockSpec((H, 128), lambda i: (0, 0)),
            pl.BlockSpec((1, 128), lambda i: (0, 0)),
            pl.BlockSpec((128, 128), lambda i: (0, 0)),
            pl.BlockSpec((1, 128), lambda i: (0, 0)),
        ],
        out_specs=pl.BlockSpec((G, 128), lambda i: (0, 0)),
        out_shape=jax.ShapeDtypeStruct((G, 128), jnp.float32),
        scratch_shapes=[pltpu.VMEM((G, H), jnp.float32)],
    )(s2, sump, sumq, cntd, cnts, b2c, batchp, wl1p, bl1p, wl2p, bl2p)


def kernel(x, edge_index, edge_weight, batch,
           W1_self, b1_self, W1_s2d, b1_s2d, W1_d2s, b1_d2s,
           W2_self, b2_self, W2_s2d, b2_s2d, W2_d2s, b2_d2s,
           Wl1, bl1, Wl2, bl2):
    src3 = edge_index[0]
    dst3 = edge_index[1]
    wcat1 = jnp.concatenate([W1_self, 0.5 * W1_s2d, 0.5 * W1_d2s], axis=1)
    b1c = (b1_self + 0.5 * b1_s2d + 0.5 * b1_d2s).reshape(1, H)
    wcat2 = jnp.concatenate([W2_self, 0.5 * W2_s2d, 0.5 * W2_d2s], axis=1)
    b2c = (b2_self + 0.5 * b2_s2d + 0.5 * b2_d2s).reshape(1, H)
    xp = jnp.pad(x, ((0, NPAD - N), (0, 0)))
    batchp = jnp.pad(batch, (0, NPAD - N), constant_values=G).reshape(NPAD, 1)
    wl1p = jnp.pad(Wl1, ((0, 0), (0, 128 - Wl1.shape[1])))
    bl1p = jnp.pad(bl1, (0, 128 - bl1.shape[0])).reshape(1, 128)
    wl2p = jnp.pad(Wl2, ((0, 128 - Wl2.shape[0]), (0, 128 - Wl2.shape[1])))
    bl2p = jnp.pad(bl2, (0, 128 - bl2.shape[0])).reshape(1, 128)

    s1, p1, q1 = _tc_transform(xp, wcat1)
    sump1, sumq1, cnt = _sc_agg(src3, dst3,
                                p1.reshape(2 * NPAD, HH),
                                q1.reshape(2 * NPAD, HH), True)
    cntd = cnt[0].reshape(NPAD, 1)
    cnts = cnt[1].reshape(NPAD, 1)
    s2, p2, q2 = _tc_combine(s1, sump1, sumq1, cntd, cnts, b1c, wcat2)
    sump2, sumq2, _ = _sc_agg(src3, dst3,
                              p2.reshape(2 * NPAD, HH),
                              q2.reshape(2 * NPAD, HH), False)
    out = _tc_final(s2, sump2, sumq2, cntd, cnts, b2c, batchp,
                    wl1p, bl1p, wl2p, bl2p)
    return out[:, :1]


# speedup vs baseline: 6.4755x; 1.1547x over previous
import functools

import jax
import jax.numpy as jnp
from jax import lax
from jax.experimental import pallas as pl
from jax.experimental.pallas import tpu as pltpu
from jax.experimental.pallas import tpu_sc as plsc

N = 10000
NPAD = 10240
E = 160000
H = 256
HH = 128
G = 64
BLK = 1024
GRID = NPAD // BLK
NTILES = 16
K = 80
EPT = E // NTILES
CPT = EPT // K
RPT = NPAD // NTILES


def _tc1_body(x_ref, w_ref, s_ref, p_ref, q_ref):
    r = jnp.dot(x_ref[...], w_ref[...], preferred_element_type=jnp.float32)
    s_ref[...] = r[:, :H]
    p_ref[...] = jnp.stack([r[:, H:H + HH], r[:, H + HH:H + 2 * HH]])
    q_ref[...] = jnp.stack([r[:, H + 2 * HH:H + 3 * HH], r[:, H + 3 * HH:]])


def _tc_transform(xp, wcat):
    return pl.pallas_call(
        _tc1_body,
        grid=(GRID,),
        in_specs=[
            pl.BlockSpec((BLK, H), lambda i: (i, 0)),
            pl.BlockSpec((H, 3 * H), lambda i: (0, 0)),
        ],
        out_specs=[
            pl.BlockSpec((BLK, H), lambda i: (i, 0)),
            pl.BlockSpec((2, BLK, HH), lambda i: (0, i, 0)),
            pl.BlockSpec((2, BLK, HH), lambda i: (0, i, 0)),
        ],
        out_shape=[
            jax.ShapeDtypeStruct((NPAD, H), jnp.float32),
            jax.ShapeDtypeStruct((2, NPAD, HH), jnp.float32),
            jax.ShapeDtypeStruct((2, NPAD, HH), jnp.float32),
        ],
    )(xp, wcat)


def _sc_agg_body(do_cnt, src_h, dst_h, p_h, q_h, sump_h, sumq_h, cnt_h,
                 gbuf0, gbuf1, gbuf2, gbuf3, sbuf0, sbuf1, sbuf2, sbuf3,
                 gadj0, gadj1, rows0, rows1,
                 onesv, zvec, acc, cacc, sem_g, sem_i, sem_s, sem_c):
    gbuf = (gbuf0, gbuf1, gbuf2, gbuf3)
    sbuf = (sbuf0, sbuf1, sbuf2, sbuf3)
    gadj = (gadj0, gadj1)
    rows = (rows0, rows1)
    c = lax.axis_index("c")
    s = lax.axis_index("s")
    row_off = c * NPAD
    base = s * RPT

    def _zero_rows0():
        def _zb(r, _):
            for j in range(HH // 16):
                rows[0][r, pl.ds(j * 16, 16)] = jnp.zeros((16,), jnp.float32)
            return 0
        lax.fori_loop(0, K, _zb, 0)

    def _zc(j, _):
        zvec[pl.ds(j * 16, 16)] = jnp.zeros((16,), jnp.float32)
        return 0
    lax.fori_loop(0, RPT // 16, _zc, 0)
    for j in range(K // 16):
        onesv[pl.ds(j * 16, 16)] = jnp.ones((16,), jnp.float32)

    def _zero_acc():
        for kk in range(RPT // K):
            pltpu.sync_copy(rows[0], acc.at[pl.ds(base + kk * K, K), :])

    def _run_phase(table_h, g_h, s_h, cnt_this):
        off = jnp.full((16,), row_off, jnp.int32)

        ebase = s * EPT

        def _load_idx(i, sl):
            pltpu.async_copy(g_h.at[pl.ds(ebase + i * K, K)], gbuf[sl], sem_i)
            pltpu.async_copy(s_h.at[pl.ds(ebase + i * K, K)], sbuf[sl], sem_i)

        def _wait_idx(i, sl):
            pltpu.make_async_copy(
                g_h.at[pl.ds(ebase + i * K, K)], gbuf[sl], sem_i).wait()
            pltpu.make_async_copy(
                s_h.at[pl.ds(ebase + i * K, K)], sbuf[sl], sem_i).wait()

        def _adj_and_gather(i, sl, b):
            for j in range(K // 16):
                gadj[b][pl.ds(j * 16, 16)] = (
                    gbuf[sl][pl.ds(j * 16, 16)] + off)
            pltpu.async_copy(table_h.at[gadj[b]], rows[b], sem_g)

        def _wait_gather(b):
            pltpu.make_async_copy(table_h.at[gadj[b]], rows[b], sem_g).wait()

        def _issue_scatter(sl, b):
            pltpu.async_copy(rows[b], acc.at[sbuf[sl]], sem_s, add=True)

        def _wait_scatter(sl, b):
            pltpu.make_async_copy(rows[b], acc.at[sbuf[sl]], sem_s).wait()

        def _issue_cnt(sl):
            @pl.when(c == 0)
            def _():
                pltpu.async_copy(onesv, cacc.at[sbuf[sl]], sem_c, add=True)

            @pl.when(c == 1)
            def _():
                pltpu.async_copy(onesv, cacc.at[gbuf[sl]], sem_c, add=True)

        def _wait_cnt(sl):
            @pl.when(c == 0)
            def _():
                pltpu.make_async_copy(onesv, cacc.at[sbuf[sl]], sem_c).wait()

            @pl.when(c == 1)
            def _():
                pltpu.make_async_copy(onesv, cacc.at[gbuf[sl]], sem_c).wait()

        def _step(j, sl, b, first=False, last=False):
            if not last:
                _wait_idx(j + 1, (sl + 1) % 4)
            if not first:
                _wait_scatter((sl - 1) % 4, 1 - b)
                if cnt_this:
                    _wait_cnt((sl - 1) % 4)
            if not last:
                _adj_and_gather(j + 1, (sl + 1) % 4, 1 - b)
            _wait_gather(b)
            _issue_scatter(sl, b)
            if cnt_this:
                _issue_cnt(sl)
            if not last:
                _load_idx(jnp.minimum(j + 2, CPT - 1), (sl + 2) % 4)

        _load_idx(0, 0)
        _wait_idx(0, 0)
        _load_idx(1, 1)
        _adj_and_gather(0, 0, 0)

        _step(0, 0, 0, first=True)

        def _quad(j, carry):
            _step(4 * j + 1, 1, 1)
            _step(4 * j + 2, 2, 0)
            _step(4 * j + 3, 3, 1)
            _step(4 * j + 4, 0, 0)
            return carry
        lax.fori_loop(0, (CPT - 5) // 4, _quad, 0)
        _step(CPT - 4, 1, 1)
        _step(CPT - 3, 2, 0)
        _step(CPT - 2, 3, 1)
        _step(CPT - 1, 0, 0, last=True)
        _wait_scatter(0, 0)
        if cnt_this:
            _wait_cnt(0)
        _wait_idx(CPT - 1, 1)

    _zero_rows0()
    _zero_acc()
    if do_cnt:
        pltpu.sync_copy(zvec, cacc.at[pl.ds(base, RPT)])
    plsc.subcore_barrier()
    _run_phase(p_h, src_h, dst_h, do_cnt)
    plsc.subcore_barrier()
    pltpu.sync_copy(acc.at[pl.ds(base, RPT), :],
                    sump_h.at[c, pl.ds(base, RPT), :])
    if do_cnt:
        pltpu.sync_copy(cacc.at[pl.ds(base, RPT)],
                        cnt_h.at[c, pl.ds(base, RPT)])

    _zero_rows0()
    _zero_acc()
    plsc.subcore_barrier()
    _run_phase(q_h, dst_h, src_h, False)
    plsc.subcore_barrier()
    pltpu.sync_copy(acc.at[pl.ds(base, RPT), :],
                    sumq_h.at[c, pl.ds(base, RPT), :])


def _sc_agg(src3, dst3, p2d, q2d, do_cnt):
    mesh = plsc.VectorSubcoreMesh(core_axis_name="c", subcore_axis_name="s")
    f = pl.kernel(
        functools.partial(_sc_agg_body, do_cnt),
        out_type=[
            jax.ShapeDtypeStruct((2, NPAD, HH), jnp.float32),
            jax.ShapeDtypeStruct((2, NPAD, HH), jnp.float32),
            jax.ShapeDtypeStruct((2, NPAD), jnp.float32),
        ],
        mesh=mesh,
        scratch_types=(
            [pltpu.VMEM((K,), jnp.int32)] * 4 +
            [pltpu.VMEM((K,), jnp.int32)] * 4 +
            [pltpu.VMEM((K,), jnp.int32)] * 2 +
            [pltpu.VMEM((K, HH), jnp.float32)] * 2 +
            [
                pltpu.VMEM((K,), jnp.float32),
                pltpu.VMEM((RPT,), jnp.float32),
                pltpu.VMEM_SHARED((NPAD, HH), jnp.float32),
                pltpu.VMEM_SHARED((NPAD,), jnp.float32),
                pltpu.SemaphoreType.DMA,
                pltpu.SemaphoreType.DMA,
                pltpu.SemaphoreType.DMA,
                pltpu.SemaphoreType.DMA,
            ]
        ),
    )
    return f(src3, dst3, p2d, q2d)


def _inv_counts(cd_ref, cs_ref):
    invd = 1.0 / jnp.maximum(cd_ref[...], 1.0)
    invs = 1.0 / jnp.maximum(cs_ref[...], 1.0)
    return invd, invs


def _combine(s_ref, sp_ref, sq_ref, b_ref, invd, invs):
    sp = jnp.concatenate([sp_ref[0], sp_ref[1]], axis=1)
    sq = jnp.concatenate([sq_ref[0], sq_ref[1]], axis=1)
    return s_ref[...] + sp * invd + sq * invs + b_ref[...]


def _tc2_body(s_ref, sp_ref, sq_ref, cd_ref, cs_ref, b_ref, w_ref,
              s2_ref, p2_ref, q2_ref):
    invd, invs = _inv_counts(cd_ref, cs_ref)
    h = jnp.maximum(_combine(s_ref, sp_ref, sq_ref, b_ref, invd, invs), 0.0)
    r = jnp.dot(h, w_ref[...], preferred_element_type=jnp.float32)
    s2_ref[...] = r[:, :H]
    p2_ref[...] = jnp.stack([r[:, H:H + HH], r[:, H + HH:H + 2 * HH]])
    q2_ref[...] = jnp.stack([r[:, H + 2 * HH:H + 3 * HH], r[:, H + 3 * HH:]])


def _tc_combine(s1, sump, sumq, cntd, cnts, b1c, wcat2):
    return pl.pallas_call(
        _tc2_body,
        grid=(GRID,),
        in_specs=[
            pl.BlockSpec((BLK, H), lambda i: (i, 0)),
            pl.BlockSpec((2, BLK, HH), lambda i: (0, i, 0)),
            pl.BlockSpec((2, BLK, HH), lambda i: (0, i, 0)),
            pl.BlockSpec((BLK, 1), lambda i: (i, 0)),
            pl.BlockSpec((BLK, 1), lambda i: (i, 0)),
            pl.BlockSpec((1, H), lambda i: (0, 0)),
            pl.BlockSpec((H, 3 * H), lambda i: (0, 0)),
        ],
        out_specs=[
            pl.BlockSpec((BLK, H), lambda i: (i, 0)),
            pl.BlockSpec((2, BLK, HH), lambda i: (0, i, 0)),
            pl.BlockSpec((2, BLK, HH), lambda i: (0, i, 0)),
        ],
        out_shape=[
            jax.ShapeDtypeStruct((NPAD, H), jnp.float32),
            jax.ShapeDtypeStruct((2, NPAD, HH), jnp.float32),
            jax.ShapeDtypeStruct((2, NPAD, HH), jnp.float32),
        ],
    )(s1, sump, sumq, cntd, cnts, b1c, wcat2)


def _tc3_body(s_ref, sp_ref, sq_ref, cd_ref, cs_ref, b_ref, batch_ref,
              wl1_ref, bl1_ref, wl2_ref, bl2_ref, out_ref, pool_ref):
    i = pl.program_id(0)
    invd, invs = _inv_counts(cd_ref, cs_ref)
    h = _combine(s_ref, sp_ref, sq_ref, b_ref, invd, invs)

    @pl.when(i == 0)
    def _():
        pool_ref[...] = jnp.full((G, H), -jnp.inf, jnp.float32)

    bslice = batch_ref[...]
    glo = jnp.min(bslice)
    ghi = jnp.minimum(jnp.max(bslice), G - 1)

    def seg(g, carry):
        m = bslice == g
        v = jnp.max(jnp.where(m, h, -jnp.inf), axis=0, keepdims=True)
        sel = lax.broadcasted_iota(jnp.int32, (G, 1), 0) == g
        pool_ref[...] = jnp.maximum(pool_ref[...],
                                    jnp.where(sel, v, -jnp.inf))
        return carry
    lax.fori_loop(glo, ghi + 1, seg, 0)

    @pl.when(i == GRID - 1)
    def _():
        p = pool_ref[...]
        t = jnp.maximum(
            jnp.dot(p, wl1_ref[...], preferred_element_type=jnp.float32)
            + bl1_ref[...], 0.0)
        out_ref[...] = (
            jnp.dot(t, wl2_ref[...], preferred_element_type=jnp.float32)
            + bl2_ref[...])


def _tc_final(s2, sump, sumq, cntd, cnts, b2c, batchp, wl1p, bl1p, wl2p, bl2p):
    return pl.pallas_call(
        _tc3_body,
        grid=(GRID,),
        in_specs=[
            pl.BlockSpec((BLK, H), lambda i: (i, 0)),
            pl.BlockSpec((2, BLK, HH), lambda i: (0, i, 0)),
            pl.BlockSpec((2, BLK, HH), lambda i: (0, i, 0)),
            pl.BlockSpec((BLK, 1), lambda i: (i, 0)),
            pl.BlockSpec((BLK, 1), lambda i: (i, 0)),
            pl.BlockSpec((1, H), lambda i: (0, 0)),
            pl.BlockSpec((BLK, 1), lambda i: (i, 0)),
            pl.BlockSpec((H, 128), lambda i: (0, 0)),
            pl.BlockSpec((1, 128), lambda i: (0, 0)),
            pl.BlockSpec((128, 128), lambda i: (0, 0)),
            pl.BlockSpec((1, 128), lambda i: (0, 0)),
        ],
        out_specs=pl.BlockSpec((G, 128), lambda i: (0, 0)),
        out_shape=jax.ShapeDtypeStruct((G, 128), jnp.float32),
        scratch_shapes=[pltpu.VMEM((G, H), jnp.float32)],
    )(s2, sump, sumq, cntd, cnts, b2c, batchp, wl1p, bl1p, wl2p, bl2p)


def kernel(x, edge_index, edge_weight, batch,
           W1_self, b1_self, W1_s2d, b1_s2d, W1_d2s, b1_d2s,
           W2_self, b2_self, W2_s2d, b2_s2d, W2_d2s, b2_d2s,
           Wl1, bl1, Wl2, bl2):
    src3 = edge_index[0]
    dst3 = edge_index[1]
    wcat1 = jnp.concatenate([W1_self, 0.5 * W1_s2d, 0.5 * W1_d2s], axis=1)
    b1c = (b1_self + 0.5 * b1_s2d + 0.5 * b1_d2s).reshape(1, H)
    wcat2 = jnp.concatenate([W2_self, 0.5 * W2_s2d, 0.5 * W2_d2s], axis=1)
    b2c = (b2_self + 0.5 * b2_s2d + 0.5 * b2_d2s).reshape(1, H)
    xp = jnp.pad(x, ((0, NPAD - N), (0, 0)))
    batchp = jnp.pad(batch, (0, NPAD - N), constant_values=G).reshape(NPAD, 1)
    wl1p = jnp.pad(Wl1, ((0, 0), (0, 128 - Wl1.shape[1])))
    bl1p = jnp.pad(bl1, (0, 128 - bl1.shape[0])).reshape(1, 128)
    wl2p = jnp.pad(Wl2, ((0, 128 - Wl2.shape[0]), (0, 128 - Wl2.shape[1])))
    bl2p = jnp.pad(bl2, (0, 128 - bl2.shape[0])).reshape(1, 128)

    s1, p1, q1 = _tc_transform(xp, wcat1)
    sump1, sumq1, cnt = _sc_agg(src3, dst3,
                                p1.reshape(2 * NPAD, HH),
                                q1.reshape(2 * NPAD, HH), True)
    cntd = cnt[0].reshape(NPAD, 1)
    cnts = cnt[1].reshape(NPAD, 1)
    s2, p2, q2 = _tc_combine(s1, sump1, sumq1, cntd, cnts, b1c, wcat2)
    sump2, sumq2, _ = _sc_agg(src3, dst3,
                              p2.reshape(2 * NPAD, HH),
                              q2.reshape(2 * NPAD, HH), False)
    out = _tc_final(s2, sump2, sumq2, cntd, cnts, b2c, batchp,
                    wl1p, bl1p, wl2p, bl2p)
    return out[:, :1]


# speedup vs baseline: 6.5115x; 1.0056x over previous
import functools

import jax
import jax.numpy as jnp
from jax import lax
from jax.experimental import pallas as pl
from jax.experimental.pallas import tpu as pltpu
from jax.experimental.pallas import tpu_sc as plsc

N = 10000
NPAD = 10240
E = 160000
H = 256
HH = 128
G = 64
BLK = 1024
GRID = NPAD // BLK
NTILES = 16
K = 80
EPT = E // NTILES
CPT = EPT // K
RPT = NPAD // NTILES


def _tc1_body(x_ref, w_ref, s_ref, p_ref, q_ref):
    r = jnp.dot(x_ref[...], w_ref[...], preferred_element_type=jnp.float32)
    s_ref[...] = r[:, :H]
    p_ref[...] = jnp.stack([r[:, H:H + HH], r[:, H + HH:H + 2 * HH]])
    q_ref[...] = jnp.stack([r[:, H + 2 * HH:H + 3 * HH], r[:, H + 3 * HH:]])


def _tc_transform(xp, wcat):
    return pl.pallas_call(
        _tc1_body,
        grid=(GRID,),
        in_specs=[
            pl.BlockSpec((BLK, H), lambda i: (i, 0)),
            pl.BlockSpec((H, 3 * H), lambda i: (0, 0)),
        ],
        out_specs=[
            pl.BlockSpec((BLK, H), lambda i: (i, 0)),
            pl.BlockSpec((2, BLK, HH), lambda i: (0, i, 0)),
            pl.BlockSpec((2, BLK, HH), lambda i: (0, i, 0)),
        ],
        out_shape=[
            jax.ShapeDtypeStruct((NPAD, H), jnp.float32),
            jax.ShapeDtypeStruct((2, NPAD, HH), jnp.float32),
            jax.ShapeDtypeStruct((2, NPAD, HH), jnp.float32),
        ],
    )(xp, wcat)


def _sc_agg_body(do_cnt, src_h, dst_h, p_h, q_h, sump_h, sumq_h, cnt_h,
                 gbuf0, gbuf1, gbuf2, gbuf3, sbuf0, sbuf1, sbuf2, sbuf3,
                 rows0, rows1,
                 onesv, zvec, acc, cacc, sem_g, sem_i, sem_s, sem_c):
    gbuf = (gbuf0, gbuf1, gbuf2, gbuf3)
    sbuf = (sbuf0, sbuf1, sbuf2, sbuf3)
    rows = (rows0, rows1)
    c = lax.axis_index("c")
    s = lax.axis_index("s")
    base = s * RPT

    def _zero_rows0():
        def _zb(r, _):
            for j in range(HH // 16):
                rows[0][r, pl.ds(j * 16, 16)] = jnp.zeros((16,), jnp.float32)
            return 0
        lax.fori_loop(0, K, _zb, 0)

    def _zc(j, _):
        zvec[pl.ds(j * 16, 16)] = jnp.zeros((16,), jnp.float32)
        return 0
    lax.fori_loop(0, RPT // 16, _zc, 0)
    for j in range(K // 16):
        onesv[pl.ds(j * 16, 16)] = jnp.ones((16,), jnp.float32)

    def _zero_acc():
        for kk in range(RPT // K):
            pltpu.sync_copy(rows[0], acc.at[pl.ds(base + kk * K, K), :])

    def _run_phase(table_h, g_h, s_h, cnt_this):
        ebase = s * EPT

        def _load_idx(i, sl):
            pltpu.async_copy(g_h.at[pl.ds(ebase + i * K, K)], gbuf[sl], sem_i)
            pltpu.async_copy(s_h.at[pl.ds(ebase + i * K, K)], sbuf[sl], sem_i)

        def _wait_idx(i, sl):
            pltpu.make_async_copy(
                g_h.at[pl.ds(ebase + i * K, K)], gbuf[sl], sem_i).wait()
            pltpu.make_async_copy(
                s_h.at[pl.ds(ebase + i * K, K)], sbuf[sl], sem_i).wait()

        def _adj_and_gather(i, sl, b):
            pltpu.async_copy(table_h.at[c].at[gbuf[sl]], rows[b], sem_g)

        def _wait_gather(b):
            pltpu.make_async_copy(
                table_h.at[c].at[gbuf[0]], rows[b], sem_g).wait()

        def _issue_scatter(sl, b):
            pltpu.async_copy(rows[b], acc.at[sbuf[sl]], sem_s, add=True)

        def _wait_scatter(sl, b):
            pltpu.make_async_copy(rows[b], acc.at[sbuf[sl]], sem_s).wait()

        def _issue_cnt(sl):
            @pl.when(c == 0)
            def _():
                pltpu.async_copy(onesv, cacc.at[sbuf[sl]], sem_c, add=True)

            @pl.when(c == 1)
            def _():
                pltpu.async_copy(onesv, cacc.at[gbuf[sl]], sem_c, add=True)

        def _wait_cnt(sl):
            @pl.when(c == 0)
            def _():
                pltpu.make_async_copy(onesv, cacc.at[sbuf[sl]], sem_c).wait()

            @pl.when(c == 1)
            def _():
                pltpu.make_async_copy(onesv, cacc.at[gbuf[sl]], sem_c).wait()

        def _step(j, sl, b, first=False, last=False):
            if not last:
                _wait_idx(j + 1, (sl + 1) % 4)
            if not first:
                _wait_scatter((sl - 1) % 4, 1 - b)
                if cnt_this:
                    _wait_cnt((sl - 1) % 4)
            if not last:
                _adj_and_gather(j + 1, (sl + 1) % 4, 1 - b)
            _wait_gather(b)
            _issue_scatter(sl, b)
            if cnt_this:
                _issue_cnt(sl)
            if not last:
                _load_idx(jnp.minimum(j + 2, CPT - 1), (sl + 2) % 4)

        _load_idx(0, 0)
        _wait_idx(0, 0)
        _load_idx(1, 1)
        _adj_and_gather(0, 0, 0)

        _step(0, 0, 0, first=True)

        def _quad(j, carry):
            _step(4 * j + 1, 1, 1)
            _step(4 * j + 2, 2, 0)
            _step(4 * j + 3, 3, 1)
            _step(4 * j + 4, 0, 0)
            return carry
        lax.fori_loop(0, (CPT - 5) // 4, _quad, 0)
        _step(CPT - 4, 1, 1)
        _step(CPT - 3, 2, 0)
        _step(CPT - 2, 3, 1)
        _step(CPT - 1, 0, 0, last=True)
        _wait_scatter(0, 0)
        if cnt_this:
            _wait_cnt(0)
        _wait_idx(CPT - 1, 1)

    _zero_rows0()
    _zero_acc()
    if do_cnt:
        pltpu.sync_copy(zvec, cacc.at[pl.ds(base, RPT)])
    plsc.subcore_barrier()
    _run_phase(p_h, src_h, dst_h, do_cnt)
    plsc.subcore_barrier()
    pltpu.sync_copy(acc.at[pl.ds(base, RPT), :],
                    sump_h.at[c, pl.ds(base, RPT), :])
    if do_cnt:
        pltpu.sync_copy(cacc.at[pl.ds(base, RPT)],
                        cnt_h.at[c, pl.ds(base, RPT)])

    _zero_rows0()
    _zero_acc()
    plsc.subcore_barrier()
    _run_phase(q_h, dst_h, src_h, False)
    plsc.subcore_barrier()
    pltpu.sync_copy(acc.at[pl.ds(base, RPT), :],
                    sumq_h.at[c, pl.ds(base, RPT), :])


def _sc_agg(src3, dst3, p2d, q2d, do_cnt):
    mesh = plsc.VectorSubcoreMesh(core_axis_name="c", subcore_axis_name="s")
    f = pl.kernel(
        functools.partial(_sc_agg_body, do_cnt),
        out_type=[
            jax.ShapeDtypeStruct((2, NPAD, HH), jnp.float32),
            jax.ShapeDtypeStruct((2, NPAD, HH), jnp.float32),
            jax.ShapeDtypeStruct((2, NPAD), jnp.float32),
        ],
        mesh=mesh,
        scratch_types=(
            [pltpu.VMEM((K,), jnp.int32)] * 4 +
            [pltpu.VMEM((K,), jnp.int32)] * 4 +
            [pltpu.VMEM((K, HH), jnp.float32)] * 2 +
            [
                pltpu.VMEM((K,), jnp.float32),
                pltpu.VMEM((RPT,), jnp.float32),
                pltpu.VMEM_SHARED((NPAD, HH), jnp.float32),
                pltpu.VMEM_SHARED((NPAD,), jnp.float32),
                pltpu.SemaphoreType.DMA,
                pltpu.SemaphoreType.DMA,
                pltpu.SemaphoreType.DMA,
                pltpu.SemaphoreType.DMA,
            ]
        ),
    )
    return f(src3, dst3, p2d, q2d)


def _inv_counts(cd_ref, cs_ref):
    invd = 1.0 / jnp.maximum(cd_ref[...], 1.0)
    invs = 1.0 / jnp.maximum(cs_ref[...], 1.0)
    return invd, invs


def _combine(s_ref, sp_ref, sq_ref, b_ref, invd, invs):
    sp = jnp.concatenate([sp_ref[0], sp_ref[1]], axis=1)
    sq = jnp.concatenate([sq_ref[0], sq_ref[1]], axis=1)
    return s_ref[...] + sp * invd + sq * invs + b_ref[...]


def _tc2_body(s_ref, sp_ref, sq_ref, cd_ref, cs_ref, b_ref, w_ref,
              s2_ref, p2_ref, q2_ref):
    invd, invs = _inv_counts(cd_ref, cs_ref)
    h = jnp.maximum(_combine(s_ref, sp_ref, sq_ref, b_ref, invd, invs), 0.0)
    r = jnp.dot(h, w_ref[...], preferred_element_type=jnp.float32)
    s2_ref[...] = r[:, :H]
    p2_ref[...] = jnp.stack([r[:, H:H + HH], r[:, H + HH:H + 2 * HH]])
    q2_ref[...] = jnp.stack([r[:, H + 2 * HH:H + 3 * HH], r[:, H + 3 * HH:]])


def _tc_combine(s1, sump, sumq, cntd, cnts, b1c, wcat2):
    return pl.pallas_call(
        _tc2_body,
        grid=(GRID,),
        in_specs=[
            pl.BlockSpec((BLK, H), lambda i: (i, 0)),
            pl.BlockSpec((2, BLK, HH), lambda i: (0, i, 0)),
            pl.BlockSpec((2, BLK, HH), lambda i: (0, i, 0)),
            pl.BlockSpec((BLK, 1), lambda i: (i, 0)),
            pl.BlockSpec((BLK, 1), lambda i: (i, 0)),
            pl.BlockSpec((1, H), lambda i: (0, 0)),
            pl.BlockSpec((H, 3 * H), lambda i: (0, 0)),
        ],
        out_specs=[
            pl.BlockSpec((BLK, H), lambda i: (i, 0)),
            pl.BlockSpec((2, BLK, HH), lambda i: (0, i, 0)),
            pl.BlockSpec((2, BLK, HH), lambda i: (0, i, 0)),
        ],
        out_shape=[
            jax.ShapeDtypeStruct((NPAD, H), jnp.float32),
            jax.ShapeDtypeStruct((2, NPAD, HH), jnp.float32),
            jax.ShapeDtypeStruct((2, NPAD, HH), jnp.float32),
        ],
    )(s1, sump, sumq, cntd, cnts, b1c, wcat2)


def _tc3_body(s_ref, sp_ref, sq_ref, cd_ref, cs_ref, b_ref, batch_ref,
              wl1_ref, bl1_ref, wl2_ref, bl2_ref, out_ref, pool_ref):
    i = pl.program_id(0)
    invd, invs = _inv_counts(cd_ref, cs_ref)
    h = _combine(s_ref, sp_ref, sq_ref, b_ref, invd, invs)

    @pl.when(i == 0)
    def _():
        pool_ref[...] = jnp.full((G, H), -jnp.inf, jnp.float32)

    bslice = batch_ref[...]
    glo = jnp.min(bslice)
    ghi = jnp.minimum(jnp.max(bslice), G - 1)

    def seg(g, carry):
        m = bslice == g
        v = jnp.max(jnp.where(m, h, -jnp.inf), axis=0, keepdims=True)
        sel = lax.broadcasted_iota(jnp.int32, (G, 1), 0) == g
        pool_ref[...] = jnp.maximum(pool_ref[...],
                                    jnp.where(sel, v, -jnp.inf))
        return carry
    lax.fori_loop(glo, ghi + 1, seg, 0)

    @pl.when(i == GRID - 1)
    def _():
        p = pool_ref[...]
        t = jnp.maximum(
            jnp.dot(p, wl1_ref[...], preferred_element_type=jnp.float32)
            + bl1_ref[...], 0.0)
        out_ref[...] = (
            jnp.dot(t, wl2_ref[...], preferred_element_type=jnp.float32)
            + bl2_ref[...])


def _tc_final(s2, sump, sumq, cntd, cnts, b2c, batchp, wl1p, bl1p, wl2p, bl2p):
    return pl.pallas_call(
        _tc3_body,
        grid=(GRID,),
        in_specs=[
            pl.BlockSpec((BLK, H), lambda i: (i, 0)),
            pl.BlockSpec((2, BLK, HH), lambda i: (0, i, 0)),
            pl.BlockSpec((2, BLK, HH), lambda i: (0, i, 0)),
            pl.BlockSpec((BLK, 1), lambda i: (i, 0)),
            pl.BlockSpec((BLK, 1), lambda i: (i, 0)),
            pl.BlockSpec((1, H), lambda i: (0, 0)),
            pl.BlockSpec((BLK, 1), lambda i: (i, 0)),
            pl.BlockSpec((H, 128), lambda i: (0, 0)),
            pl.BlockSpec((1, 128), lambda i: (0, 0)),
            pl.BlockSpec((128, 128), lambda i: (0, 0)),
            pl.BlockSpec((1, 128), lambda i: (0, 0)),
        ],
        out_specs=pl.BlockSpec((G, 128), lambda i: (0, 0)),
        out_shape=jax.ShapeDtypeStruct((G, 128), jnp.float32),
        scratch_shapes=[pltpu.VMEM((G, H), jnp.float32)],
    )(s2, sump, sumq, cntd, cnts, b2c, batchp, wl1p, bl1p, wl2p, bl2p)


def kernel(x, edge_index, edge_weight, batch,
           W1_self, b1_self, W1_s2d, b1_s2d, W1_d2s, b1_d2s,
           W2_self, b2_self, W2_s2d, b2_s2d, W2_d2s, b2_d2s,
           Wl1, bl1, Wl2, bl2):
    src3 = edge_index[0]
    dst3 = edge_index[1]
    wcat1 = jnp.concatenate([W1_self, 0.5 * W1_s2d, 0.5 * W1_d2s], axis=1)
    b1c = (b1_self + 0.5 * b1_s2d + 0.5 * b1_d2s).reshape(1, H)
    wcat2 = jnp.concatenate([W2_self, 0.5 * W2_s2d, 0.5 * W2_d2s], axis=1)
    b2c = (b2_self + 0.5 * b2_s2d + 0.5 * b2_d2s).reshape(1, H)
    xp = jnp.pad(x, ((0, NPAD - N), (0, 0)))
    batchp = jnp.pad(batch, (0, NPAD - N), constant_values=G).reshape(NPAD, 1)
    wl1p = jnp.pad(Wl1, ((0, 0), (0, 128 - Wl1.shape[1])))
    bl1p = jnp.pad(bl1, (0, 128 - bl1.shape[0])).reshape(1, 128)
    wl2p = jnp.pad(Wl2, ((0, 128 - Wl2.shape[0]), (0, 128 - Wl2.shape[1])))
    bl2p = jnp.pad(bl2, (0, 128 - bl2.shape[0])).reshape(1, 128)

    s1, p1, q1 = _tc_transform(xp, wcat1)
    sump1, sumq1, cnt = _sc_agg(src3, dst3, p1, q1, True)
    cntd = cnt[0].reshape(NPAD, 1)
    cnts = cnt[1].reshape(NPAD, 1)
    s2, p2, q2 = _tc_combine(s1, sump1, sumq1, cntd, cnts, b1c, wcat2)
    sump2, sumq2, _ = _sc_agg(src3, dst3, p2, q2, False)
    out = _tc_final(s2, sump2, sumq2, cntd, cnts, b2c, batchp,
                    wl1p, bl1p, wl2p, bl2p)
    return out[:, :1]


# speedup vs baseline: 6.5374x; 1.0040x over previous
import functools

import jax
import jax.numpy as jnp
from jax import lax
from jax.experimental import pallas as pl
from jax.experimental.pallas import tpu as pltpu
from jax.experimental.pallas import tpu_sc as plsc

N = 10000
NPAD = 10000
CPAD = 10240
E = 160000
H = 256
HH = 128
G = 64
BLK = 1000
GRID = NPAD // BLK
NTILES = 16
K = 80
EPT = E // NTILES
CPT = EPT // K
RPT = NPAD // NTILES
RPTC = CPAD // NTILES


def _tc1_body(x_ref, w_ref, s_ref, p_ref, q_ref):
    r = jnp.dot(x_ref[...], w_ref[...], preferred_element_type=jnp.float32)
    s_ref[...] = r[:, :H]
    p_ref[...] = jnp.stack([r[:, H:H + HH], r[:, H + HH:H + 2 * HH]])
    q_ref[...] = jnp.stack([r[:, H + 2 * HH:H + 3 * HH], r[:, H + 3 * HH:]])


def _tc_transform(xp, wcat):
    return pl.pallas_call(
        _tc1_body,
        grid=(GRID,),
        in_specs=[
            pl.BlockSpec((BLK, H), lambda i: (i, 0)),
            pl.BlockSpec((H, 3 * H), lambda i: (0, 0)),
        ],
        out_specs=[
            pl.BlockSpec((BLK, H), lambda i: (i, 0)),
            pl.BlockSpec((2, BLK, HH), lambda i: (0, i, 0)),
            pl.BlockSpec((2, BLK, HH), lambda i: (0, i, 0)),
        ],
        out_shape=[
            jax.ShapeDtypeStruct((NPAD, H), jnp.float32),
            jax.ShapeDtypeStruct((2, NPAD, HH), jnp.float32),
            jax.ShapeDtypeStruct((2, NPAD, HH), jnp.float32),
        ],
    )(xp, wcat)


def _sc_agg_body(do_cnt, src_h, dst_h, p_h, q_h, sump_h, sumq_h, cnt_h,
                 gbuf0, gbuf1, gbuf2, gbuf3, sbuf0, sbuf1, sbuf2, sbuf3,
                 rows0, rows1, rows2,
                 onesv, zvec, acc, cacc, sem_g, sem_i, sem_s, sem_c):
    gbuf = (gbuf0, gbuf1, gbuf2, gbuf3)
    sbuf = (sbuf0, sbuf1, sbuf2, sbuf3)
    rows = (rows0, rows1, rows2)
    c = lax.axis_index("c")
    s = lax.axis_index("s")
    base = pl.multiple_of(s * 624, 8)
    basec = pl.multiple_of(s * RPTC, RPTC)

    def _zero_rows0():
        def _zb(r, _):
            for j in range(HH // 16):
                rows[0][r, pl.ds(j * 16, 16)] = jnp.zeros((16,), jnp.float32)
            return 0
        lax.fori_loop(0, K, _zb, 0)

    for j in range(128 // 16):
        zvec[pl.ds(j * 16, 16)] = jnp.zeros((16,), jnp.float32)
    for j in range(K // 16):
        onesv[pl.ds(j * 16, 16)] = jnp.ones((16,), jnp.float32)

    def _zero_acc():
        for kk in range(624 // K):
            pltpu.sync_copy(rows[0], acc.at[pl.ds(base + kk * K, K), :])
        pltpu.sync_copy(rows[0].at[pl.ds(0, 624 % K), :],
                        acc.at[pl.ds(base + 624 - 624 % K, 624 % K), :])

        @pl.when(s == NTILES - 1)
        def _():
            pltpu.sync_copy(rows[0].at[pl.ds(0, 16), :],
                            acc.at[pl.ds(NPAD - 16, 16), :])

    def _run_phase(table_h, g_h, s_h, cnt_this):
        ebase = s * EPT

        def _load_idx(i, sl):
            pltpu.async_copy(g_h.at[pl.ds(ebase + i * K, K)], gbuf[sl], sem_i)
            pltpu.async_copy(s_h.at[pl.ds(ebase + i * K, K)], sbuf[sl], sem_i)

        def _wait_idx(i, sl):
            pltpu.make_async_copy(
                g_h.at[pl.ds(ebase + i * K, K)], gbuf[sl], sem_i).wait()
            pltpu.make_async_copy(
                s_h.at[pl.ds(ebase + i * K, K)], sbuf[sl], sem_i).wait()

        def _issue_gather(sl, b):
            pltpu.async_copy(table_h.at[c].at[gbuf[sl]], rows[b], sem_g)

        def _wait_gather(b):
            pltpu.make_async_copy(
                table_h.at[c].at[gbuf[0]], rows[b], sem_g).wait()

        def _issue_scatter(sl, b):
            pltpu.async_copy(rows[b], acc.at[sbuf[sl]], sem_s, add=True)

        def _wait_scatter(sl, b):
            pltpu.make_async_copy(rows[b], acc.at[sbuf[sl]], sem_s).wait()

        def _issue_cnt(sl):
            @pl.when(c == 0)
            def _():
                pltpu.async_copy(onesv, cacc.at[sbuf[sl]], sem_c, add=True)

            @pl.when(c == 1)
            def _():
                pltpu.async_copy(onesv, cacc.at[gbuf[sl]], sem_c, add=True)

        def _wait_cnt(sl):
            @pl.when(c == 0)
            def _():
                pltpu.make_async_copy(onesv, cacc.at[sbuf[sl]], sem_c).wait()

            @pl.when(c == 1)
            def _():
                pltpu.make_async_copy(onesv, cacc.at[gbuf[sl]], sem_c).wait()

        def _step(j, sl, b, warm=True, has_next=True, load_next=True):
            if has_next:
                _wait_idx(j + 1, (sl + 1) % 4)
            if warm:
                _wait_scatter((sl + 2) % 4, (b + 1) % 3)
                if cnt_this:
                    _wait_cnt((sl + 2) % 4)
            if has_next:
                _issue_gather((sl + 1) % 4, (b + 1) % 3)
            _wait_gather(b)
            _issue_scatter(sl, b)
            if cnt_this:
                _issue_cnt(sl)
            if load_next:
                _load_idx(j + 2, (sl + 2) % 4)

        _load_idx(0, 0)
        _wait_idx(0, 0)
        _load_idx(1, 1)
        _issue_gather(0, 0)

        _step(0, 0, 0, warm=False)
        _step(1, 1, 1, warm=False)

        def _dodecad(m, carry):
            j = 2 + 12 * m
            for pp in range(12):
                _step(j + pp, (2 + pp) % 4, (2 + pp) % 3)
            return carry
        lax.fori_loop(0, (CPT - 5) // 12, _dodecad, 0)
        _step(CPT - 3, (CPT - 3) % 4, (CPT - 3) % 3)
        _step(CPT - 2, (CPT - 2) % 4, (CPT - 2) % 3, load_next=False)
        _step(CPT - 1, (CPT - 1) % 4, (CPT - 1) % 3,
              has_next=False, load_next=False)
        for j in (CPT - 2, CPT - 1):
            _wait_scatter(j % 4, j % 3)
            if cnt_this:
                _wait_cnt(j % 4)

    _zero_rows0()
    _zero_acc()
    if do_cnt:
        for r in range(RPTC // 128):
            pltpu.sync_copy(zvec, cacc.at[pl.ds(basec + r * 128, 128)])
    plsc.subcore_barrier()
    _run_phase(p_h, src_h, dst_h, do_cnt)
    plsc.subcore_barrier()
    pltpu.sync_copy(acc.at[pl.ds(base, 624), :],
                    sump_h.at[c, pl.ds(base, 624), :])

    @pl.when(s == NTILES - 1)
    def _():
        pltpu.sync_copy(acc.at[pl.ds(NPAD - 16, 16), :],
                        sump_h.at[c, pl.ds(NPAD - 16, 16), :])
    if do_cnt:
        pltpu.sync_copy(cacc.at[pl.ds(basec, RPTC)],
                        cnt_h.at[c, pl.ds(basec, RPTC)])

    _zero_rows0()
    _zero_acc()
    plsc.subcore_barrier()
    _run_phase(q_h, dst_h, src_h, False)
    plsc.subcore_barrier()
    pltpu.sync_copy(acc.at[pl.ds(base, 624), :],
                    sumq_h.at[c, pl.ds(base, 624), :])

    @pl.when(s == NTILES - 1)
    def _():
        pltpu.sync_copy(acc.at[pl.ds(NPAD - 16, 16), :],
                        sumq_h.at[c, pl.ds(NPAD - 16, 16), :])


def _sc_agg(src3, dst3, p2d, q2d, do_cnt):
    mesh = plsc.VectorSubcoreMesh(core_axis_name="c", subcore_axis_name="s")
    f = pl.kernel(
        functools.partial(_sc_agg_body, do_cnt),
        out_type=[
            jax.ShapeDtypeStruct((2, NPAD, HH), jnp.float32),
            jax.ShapeDtypeStruct((2, NPAD, HH), jnp.float32),
            jax.ShapeDtypeStruct((2, CPAD), jnp.float32),
        ],
        mesh=mesh,
        scratch_types=(
            [pltpu.VMEM((K,), jnp.int32)] * 4 +
            [pltpu.VMEM((K,), jnp.int32)] * 4 +
            [pltpu.VMEM((K, HH), jnp.float32)] * 3 +
            [
                pltpu.VMEM((K,), jnp.float32),
                pltpu.VMEM((128,), jnp.float32),
                pltpu.VMEM_SHARED((NPAD, HH), jnp.float32),
                pltpu.VMEM_SHARED((CPAD,), jnp.float32),
                pltpu.SemaphoreType.DMA,
                pltpu.SemaphoreType.DMA,
                pltpu.SemaphoreType.DMA,
                pltpu.SemaphoreType.DMA,
            ]
        ),
    )
    return f(src3, dst3, p2d, q2d)


def _inv_counts(cd_ref, cs_ref):
    invd = 1.0 / jnp.maximum(cd_ref[...], 1.0)
    invs = 1.0 / jnp.maximum(cs_ref[...], 1.0)
    return invd, invs


def _combine(s_ref, sp_ref, sq_ref, b_ref, invd, invs):
    sp = jnp.concatenate([sp_ref[0], sp_ref[1]], axis=1)
    sq = jnp.concatenate([sq_ref[0], sq_ref[1]], axis=1)
    return s_ref[...] + sp * invd + sq * invs + b_ref[...]


def _tc2_body(s_ref, sp_ref, sq_ref, cd_ref, cs_ref, b_ref, w_ref,
              s2_ref, p2_ref, q2_ref):
    invd, invs = _inv_counts(cd_ref, cs_ref)
    h = jnp.maximum(_combine(s_ref, sp_ref, sq_ref, b_ref, invd, invs), 0.0)
    r = jnp.dot(h, w_ref[...], preferred_element_type=jnp.float32)
    s2_ref[...] = r[:, :H]
    p2_ref[...] = jnp.stack([r[:, H:H + HH], r[:, H + HH:H + 2 * HH]])
    q2_ref[...] = jnp.stack([r[:, H + 2 * HH:H + 3 * HH], r[:, H + 3 * HH:]])


def _tc_combine(s1, sump, sumq, cntd, cnts, b1c, wcat2):
    return pl.pallas_call(
        _tc2_body,
        grid=(GRID,),
        in_specs=[
            pl.BlockSpec((BLK, H), lambda i: (i, 0)),
            pl.BlockSpec((2, BLK, HH), lambda i: (0, i, 0)),
            pl.BlockSpec((2, BLK, HH), lambda i: (0, i, 0)),
            pl.BlockSpec((BLK, 1), lambda i: (i, 0)),
            pl.BlockSpec((BLK, 1), lambda i: (i, 0)),
            pl.BlockSpec((1, H), lambda i: (0, 0)),
            pl.BlockSpec((H, 3 * H), lambda i: (0, 0)),
        ],
        out_specs=[
            pl.BlockSpec((BLK, H), lambda i: (i, 0)),
            pl.BlockSpec((2, BLK, HH), lambda i: (0, i, 0)),
            pl.BlockSpec((2, BLK, HH), lambda i: (0, i, 0)),
        ],
        out_shape=[
            jax.ShapeDtypeStruct((NPAD, H), jnp.float32),
            jax.ShapeDtypeStruct((2, NPAD, HH), jnp.float32),
            jax.ShapeDtypeStruct((2, NPAD, HH), jnp.float32),
        ],
    )(s1, sump, sumq, cntd, cnts, b1c, wcat2)


def _tc3_body(s_ref, sp_ref, sq_ref, cd_ref, cs_ref, b_ref, batch_ref,
              wl1_ref, bl1_ref, wl2_ref, bl2_ref, out_ref, pool_ref):
    i = pl.program_id(0)
    invd, invs = _inv_counts(cd_ref, cs_ref)
    h = _combine(s_ref, sp_ref, sq_ref, b_ref, invd, invs)

    @pl.when(i == 0)
    def _():
        pool_ref[...] = jnp.full((G, H), -jnp.inf, jnp.float32)

    bslice = batch_ref[...]
    glo = jnp.min(bslice)
    ghi = jnp.minimum(jnp.max(bslice), G - 1)

    def seg(g, carry):
        m = bslice == g
        v = jnp.max(jnp.where(m, h, -jnp.inf), axis=0, keepdims=True)
        sel = lax.broadcasted_iota(jnp.int32, (G, 1), 0) == g
        pool_ref[...] = jnp.maximum(pool_ref[...],
                                    jnp.where(sel, v, -jnp.inf))
        return carry
    lax.fori_loop(glo, ghi + 1, seg, 0)

    @pl.when(i == GRID - 1)
    def _():
        p = pool_ref[...]
        t = jnp.maximum(
            jnp.dot(p, wl1_ref[...], preferred_element_type=jnp.float32)
            + bl1_ref[...], 0.0)
        out_ref[...] = (
            jnp.dot(t, wl2_ref[...], preferred_element_type=jnp.float32)
            + bl2_ref[...])


def _tc_final(s2, sump, sumq, cntd, cnts, b2c, batchp, wl1p, bl1p, wl2p, bl2p):
    return pl.pallas_call(
        _tc3_body,
        grid=(GRID,),
        in_specs=[
            pl.BlockSpec((BLK, H), lambda i: (i, 0)),
            pl.BlockSpec((2, BLK, HH), lambda i: (0, i, 0)),
            pl.BlockSpec((2, BLK, HH), lambda i: (0, i, 0)),
            pl.BlockSpec((BLK, 1), lambda i: (i, 0)),
            pl.BlockSpec((BLK, 1), lambda i: (i, 0)),
            pl.BlockSpec((1, H), lambda i: (0, 0)),
            pl.BlockSpec((BLK, 1), lambda i: (i, 0)),
            pl.BlockSpec((H, 128), lambda i: (0, 0)),
            pl.BlockSpec((1, 128), lambda i: (0, 0)),
            pl.BlockSpec((128, 128), lambda i: (0, 0)),
            pl.BlockSpec((1, 128), lambda i: (0, 0)),
        ],
        out_specs=pl.BlockSpec((G, 128), lambda i: (0, 0)),
        out_shape=jax.ShapeDtypeStruct((G, 128), jnp.float32),
        scratch_shapes=[pltpu.VMEM((G, H), jnp.float32)],
    )(s2, sump, sumq, cntd, cnts, b2c, batchp, wl1p, bl1p, wl2p, bl2p)


def kernel(x, edge_index, edge_weight, batch,
           W1_self, b1_self, W1_s2d, b1_s2d, W1_d2s, b1_d2s,
           W2_self, b2_self, W2_s2d, b2_s2d, W2_d2s, b2_d2s,
           Wl1, bl1, Wl2, bl2):
    src3 = edge_index[0]
    dst3 = edge_index[1]
    wcat1 = jnp.concatenate([W1_self, 0.5 * W1_s2d, 0.5 * W1_d2s], axis=1)
    b1c = (b1_self + 0.5 * b1_s2d + 0.5 * b1_d2s).reshape(1, H)
    wcat2 = jnp.concatenate([W2_self, 0.5 * W2_s2d, 0.5 * W2_d2s], axis=1)
    b2c = (b2_self + 0.5 * b2_s2d + 0.5 * b2_d2s).reshape(1, H)
    xp = x
    batchp = batch.reshape(NPAD, 1)
    wl1p = jnp.pad(Wl1, ((0, 0), (0, 128 - Wl1.shape[1])))
    bl1p = jnp.pad(bl1, (0, 128 - bl1.shape[0])).reshape(1, 128)
    wl2p = jnp.pad(Wl2, ((0, 128 - Wl2.shape[0]), (0, 128 - Wl2.shape[1])))
    bl2p = jnp.pad(bl2, (0, 128 - bl2.shape[0])).reshape(1, 128)

    s1, p1, q1 = _tc_transform(xp, wcat1)
    sump1, sumq1, cnt = _sc_agg(src3, dst3, p1, q1, True)
    cntd = cnt[0, :NPAD].reshape(NPAD, 1)
    cnts = cnt[1, :NPAD].reshape(NPAD, 1)
    s2, p2, q2 = _tc_combine(s1, sump1, sumq1, cntd, cnts, b1c, wcat2)
    sump2, sumq2, _ = _sc_agg(src3, dst3, p2, q2, False)
    out = _tc_final(s2, sump2, sumq2, cntd, cnts, b2c, batchp,
                    wl1p, bl1p, wl2p, bl2p)
    return out[:, :1]
